# baseline (speedup 1.0000x reference)
import functools
import math

import jax
import jax.numpy as jnp
from jax import lax
from jax.experimental import pallas as pl
from jax.experimental.pallas import tpu as pltpu
from jax.experimental.pallas import tpu_sc as plsc

D_MODEL = 64
SEQ = 200
BATCH = 1024
ROWS = BATCH * SEQ
CHUNK = 40
SCALE = math.sqrt(D_MODEL)
NVEC = D_MODEL // 16


def _sc_body(nw, nchunk, idx_hbm, table_hbm, pe_hbm, out_hbm,
             idx_v, pe_v, rows_v, gsem):
    nc = nw // 16
    wid = lax.axis_index("s") * nc + lax.axis_index("c")
    base_chunk = wid * nchunk

    pltpu.sync_copy(idx_hbm.at[pl.ds(base_chunk, nchunk)], idx_v)
    pltpu.sync_copy(pe_hbm, pe_v)

    def chunk_body(c, _):
        pltpu.async_copy(table_hbm.at[idx_v.at[c]], rows_v, gsem).wait()
        s0 = lax.rem(c * CHUNK, SEQ)

        def row_body(r, _):
            for d in range(NVEC):
                sl = pl.ds(d * 16, 16)
                rows_v[r, sl] = rows_v[r, sl] * SCALE + pe_v[s0 + r, sl]
            return 0

        lax.fori_loop(0, CHUNK, row_body, 0)
        pltpu.sync_copy(rows_v, out_hbm.at[pl.ds((base_chunk + c) * CHUNK, CHUNK)])
        return 0

    lax.fori_loop(0, nchunk, chunk_body, 0)


def kernel(x, table, pe):
    info = plsc.get_sparse_core_info()
    nw = info.num_cores * info.num_subcores
    nchunk = ROWS // (nw * CHUNK)

    idx = x.astype(jnp.int32).reshape(nw * nchunk, CHUNK)
    pe2 = pe[:SEQ, 0, :]

    mesh = plsc.VectorSubcoreMesh(core_axis_name="c", subcore_axis_name="s")
    f = pl.kernel(
        functools.partial(_sc_body, nw, nchunk),
        mesh=mesh,
        compiler_params=pltpu.CompilerParams(use_tc_tiling_on_sc=False),
        out_type=jax.ShapeDtypeStruct((ROWS, D_MODEL), jnp.float32),
        scratch_types=[
            pltpu.VMEM((nchunk, CHUNK), jnp.int32),
            pltpu.VMEM((SEQ, D_MODEL), jnp.float32),
            pltpu.VMEM((CHUNK, D_MODEL), jnp.float32),
            pltpu.SemaphoreType.DMA,
        ],
    )
    out = f(idx, table, pe2)
    return out.reshape(BATCH, SEQ, D_MODEL)


# speedup vs baseline: 1.2860x; 1.2860x over previous
import functools
import math

import jax
import jax.numpy as jnp
from jax import lax
from jax.experimental import pallas as pl
from jax.experimental.pallas import tpu as pltpu
from jax.experimental.pallas import tpu_sc as plsc

D_MODEL = 64
SEQ = 200
BATCH = 1024
ROWS = BATCH * SEQ
CHUNK = 40
NBUF = 4
SCALE = math.sqrt(D_MODEL)
NVEC = D_MODEL // 16


def _sc_body(nw, nchunk, idx_hbm, table_hbm, pe_hbm, out_hbm,
             idx_v, pe_v, rows_v, obuf_v, *sems):
    gsem = sems[:NBUF]
    ssem = sems[NBUF:]
    nc = nw // 16
    wid = lax.axis_index("s") * nc + lax.axis_index("c")
    base_chunk = wid * nchunk

    pltpu.sync_copy(idx_hbm.at[pl.ds(base_chunk, nchunk)], idx_v)
    pltpu.sync_copy(pe_hbm, pe_v)

    def gather(c, b):
        return pltpu.make_async_copy(
            table_hbm.at[idx_v.at[c]], rows_v.at[b], gsem[b])

    def scatter(c, b):
        return pltpu.make_async_copy(
            obuf_v.at[b], out_hbm.at[pl.ds((base_chunk + c) * CHUNK, CHUNK)],
            ssem[b])

    for b in range(NBUF):
        gather(b, b).start()

    ngroup = nchunk // NBUF

    def group_body(g, _):
        for b in range(NBUF):
            c = g * NBUF + b
            gather(c, b).wait()

            @pl.when(g > 0)
            def _():
                scatter(c - NBUF, b).wait()

            s0 = lax.rem(c * CHUNK, SEQ)

            @plsc.parallel_loop(0, CHUNK)
            def _(r):
                for d in range(NVEC):
                    sl = pl.ds(d * 16, 16)
                    obuf_v[b, r, sl] = rows_v[b, r, sl] * SCALE + pe_v[s0 + r, sl]

            scatter(c, b).start()

            @pl.when(g < ngroup - 1)
            def _():
                gather(c + NBUF, b).start()
        return 0

    lax.fori_loop(0, ngroup, group_body, 0)

    for b in range(NBUF):
        scatter(nchunk - NBUF + b, b).wait()


def kernel(x, table, pe):
    info = plsc.get_sparse_core_info()
    nw = info.num_cores * info.num_subcores
    nchunk = ROWS // (nw * CHUNK)

    idx = x.astype(jnp.int32).reshape(nw * nchunk, CHUNK)
    pe2 = pe[:SEQ, 0, :]

    mesh = plsc.VectorSubcoreMesh(core_axis_name="c", subcore_axis_name="s")
    f = pl.kernel(
        functools.partial(_sc_body, nw, nchunk),
        mesh=mesh,
        compiler_params=pltpu.CompilerParams(use_tc_tiling_on_sc=False),
        out_type=jax.ShapeDtypeStruct((ROWS, D_MODEL), jnp.float32),
        scratch_types=[
            pltpu.VMEM((nchunk, CHUNK), jnp.int32),
            pltpu.VMEM((SEQ, D_MODEL), jnp.float32),
            pltpu.VMEM((NBUF, CHUNK, D_MODEL), jnp.float32),
            pltpu.VMEM((NBUF, CHUNK, D_MODEL), jnp.float32),
        ] + [pltpu.SemaphoreType.DMA] * (2 * NBUF),
    )
    out = f(idx, table, pe2)
    return out.reshape(BATCH, SEQ, D_MODEL)
